# baseline (speedup 1.0000x reference)
import functools

import jax
import jax.numpy as jnp
from jax import lax
from jax.experimental import pallas as pl
from jax.experimental.pallas import tpu as pltpu
from jax.experimental.pallas import tpu_sc as plsc


def _tokenize(xcat_t, xnum_t, tab_t, num_weight, num_bias):
    N_CAT, B = xcat_t.shape
    N_NUM = xnum_t.shape[0]
    _, D, VOCAB = tab_t.shape
    N_TOK = N_CAT + N_NUM

    info = plsc.get_sparse_core_info()
    NC, NS = info.num_cores, info.num_subcores
    NW = NC * NS
    cat_per_w = (N_CAT * D) // NW
    num_per_w = (N_NUM * D) // NW
    n_grp = B // 16
    VH0 = ((VOCAB // 2 + 127) // 128) * 128
    VH1 = VOCAB - VH0

    mesh = plsc.VectorSubcoreMesh(core_axis_name="c", subcore_axis_name="s")

    @functools.partial(
        pl.kernel,
        mesh=mesh,
        compiler_params=pltpu.CompilerParams(needs_layout_passes=False),
        out_type=jax.ShapeDtypeStruct((N_TOK, D, B), jnp.float32),
        scratch_types=[
            pltpu.VMEM((VH0,), jnp.float32),
            pltpu.VMEM((VH1,), jnp.float32),
            pltpu.VMEM((B,), jnp.int32),
            pltpu.VMEM((B,), jnp.float32),
            pltpu.VMEM((B,), jnp.float32),
            pltpu.VMEM((B,), jnp.float32),
            pltpu.VMEM((B,), jnp.float32),
            pltpu.VMEM((N_NUM, D), jnp.float32),
            pltpu.VMEM((N_NUM, D), jnp.float32),
            pltpu.SemaphoreType.DMA,
            pltpu.SemaphoreType.DMA,
            pltpu.SemaphoreType.DMA,
            pltpu.SemaphoreType.DMA,
            pltpu.SemaphoreType.DMA,
        ],
    )
    def body(xcat_hbm, xnum_hbm, tab_hbm, w_hbm, bias_hbm, out_hbm,
             h0, h1, idx_v, o0, o1, o2, xr_v, w_v, b_v,
             semA, semB, semO0, semO1, semO2):
        wid = lax.axis_index("s") * NC + lax.axis_index("c")
        p0 = wid * cat_per_w
        ovs = (o0, o1)
        sems = (semO0, semO1)

        def cpA(p):
            f = p // D
            d = p - f * D
            return pltpu.make_async_copy(
                tab_hbm.at[f, d, pl.ds(0, VH0)], h0, semA)

        def cpB(p):
            f = p // D
            d = p - f * D
            return pltpu.make_async_copy(
                tab_hbm.at[f, d, pl.ds(VH0, VH1)], h1, semB)

        def cpO(slot, t, d):
            return pltpu.make_async_copy(ovs[slot], out_hbm.at[t, d], sems[slot])

        cpA(p0).start()
        cpB(p0).start()
        pltpu.sync_copy(w_hbm, w_v)
        pltpu.sync_copy(bias_hbm, b_v)

        pltpu.sync_copy(xnum_hbm.at[(wid * num_per_w) // D], xr_v)

        def cpO2(q):
            j = q // D
            d = q - j * D
            return pltpu.make_async_copy(
                o2, out_hbm.at[N_CAT + j, d], semO2)

        def num_step(k):
            q = wid * num_per_w + k

            @pl.when(k > 0)
            def _():
                cpO2(q - 1).wait()

            j = q // D
            d = q - j * D
            jv = jnp.full((16,), j, jnp.int32)
            dv = jnp.full((16,), d, jnp.int32)
            wv = plsc.load_gather(w_v, [jv, dv])
            bv = plsc.load_gather(b_v, [jv, dv])

            def g(i, c):
                sl = pl.ds(i * 16, 16)
                o2[sl] = xr_v[sl] * wv + bv
                return c

            lax.fori_loop(0, n_grp, g, 0)
            cpO2(q).start()

        def cat_pair(kk, prev_f):
            for slot in range(2):
                k = kk * 2 + slot
                p = p0 + k
                f = p // D
                d = p - f * D

                @pl.when(f != prev_f)
                def _():
                    pltpu.sync_copy(xcat_hbm.at[f], idx_v)

                prev_f = f
                ov = ovs[slot]

                @pl.when(kk > 0)
                def _():
                    pp = p - 2
                    fp = pp // D
                    dp = pp - fp * D
                    cpO(slot, fp, dp).wait()

                cpA(p).wait()
                c_vh0 = jnp.full((16,), VH0, jnp.int32)
                c_and = jnp.full((16,), (1 << 17) - 1, jnp.int32)

                def gA(i, c):
                    sl = pl.ds(i * 16, 16)
                    iv = idx_v[sl]
                    ov[sl] = plsc.load_gather(h0, [iv], mask=iv < c_vh0)
                    return c

                lax.fori_loop(0, n_grp, gA, 0)

                @pl.when(k < cat_per_w - 1)
                def _():
                    cpA(p + 1).start()

                @pl.when(k < num_per_w)
                def _():
                    num_step(k)

                cpB(p).wait()

                def gB(i, c):
                    sl = pl.ds(i * 16, 16)
                    iv = idx_v[sl]
                    m = iv >= c_vh0
                    gb = plsc.load_gather(h1, [(iv - c_vh0) & c_and], mask=m)
                    ov[sl] = jnp.where(m, gb, ov[sl])
                    return c

                lax.fori_loop(0, n_grp, gB, 0)

                @pl.when(k < cat_per_w - 1)
                def _():
                    cpB(p + 1).start()

                cpO(slot, f, d).start()
            return prev_f

        lax.fori_loop(0, cat_per_w // 2, cat_pair, jnp.int32(-1))
        for slot in range(2):
            p = p0 + cat_per_w - 2 + slot
            f = p // D
            d = p - f * D
            cpO(slot, f, d).wait()
        cpO2(wid * num_per_w + num_per_w - 1).wait()

    return body(xcat_t, xnum_t, tab_t, num_weight, num_bias)


def kernel(x_cat, x_num, cat_tables, num_weight, num_bias):
    xcat_t = x_cat.T
    xnum_t = x_num.T
    tab_t = jnp.transpose(cat_tables, (0, 2, 1))
    out_t = _tokenize(xcat_t, xnum_t, tab_t, num_weight, num_bias)
    return jnp.transpose(out_t, (2, 0, 1))
